# baseline (speedup 1.0000x reference)
import functools

import jax
import jax.numpy as jnp
from jax import lax
from jax.experimental import pallas as pl
from jax.experimental.pallas import tpu as pltpu
from jax.experimental.pallas import tpu_sc as plsc

_C = 768
_NE = 8
_BT = 128
_NB = 39
_NPAD = _NB * _BT
_GC = 0.7978845608028654


def _gelu(x):
    return 0.5 * x * (1.0 + jnp.tanh(_GC * (x + 0.044715 * x * x * x)))


def _router_body(x_ref, ce_ref, d1_ref, d2_ref, d3_ref, d4_ref, w_ref, b_ref,
                 gate1_ref, i1_ref, i2_ref, g1_ref, g2_ref):
    T = x_ref.shape[0]
    ce = jnp.broadcast_to(ce_ref[...], (T, 32))
    h = jnp.concatenate([x_ref[...], ce, d1_ref[...], d2_ref[...],
                         d3_ref[...], d4_ref[...]], axis=1)
    l = jax.lax.dot_general(h, w_ref[...], (((1,), (1,)), ((), ())),
                            preferred_element_type=jnp.float32)
    l = l + b_ref[...]

    m = jnp.max(l, axis=1, keepdims=True)
    ex = jnp.exp(l - m)
    gate1_ref[...] = ex / jnp.sum(ex, axis=1, keepdims=True)

    T = l.shape[0]
    iota = jax.lax.broadcasted_iota(jnp.int32, (T, _NE), 1)
    m1 = jnp.max(l, axis=1, keepdims=True)
    i1 = jnp.min(jnp.where(l == m1, iota, _NE), axis=1, keepdims=True)
    l2 = jnp.where(iota == i1, -jnp.inf, l)
    m2 = jnp.max(l2, axis=1, keepdims=True)
    i2 = jnp.min(jnp.where(l2 == m2, iota, _NE), axis=1, keepdims=True)
    e21 = jnp.exp(m2 - m1)
    den = 1.0 + e21
    i1_ref[...] = i1
    i2_ref[...] = i2
    g1_ref[...] = 1.0 / den
    g2_ref[...] = e21 / den


def _run_router(x2d, ce, d1, d2, d3, d4, router_w, bias):
    T = x2d.shape[0]
    L = router_w.shape[1]
    f32 = jnp.float32
    return pl.pallas_call(
        _router_body,
        grid=(1,),
        in_specs=[
            pl.BlockSpec((T, _C), lambda i: (0, 0)),
            pl.BlockSpec((1, 32), lambda i: (0, 0)),
            pl.BlockSpec((T, 192), lambda i: (0, 0)),
            pl.BlockSpec((T, 192), lambda i: (0, 0)),
            pl.BlockSpec((T, 96), lambda i: (0, 0)),
            pl.BlockSpec((T, 96), lambda i: (0, 0)),
            pl.BlockSpec((_NE, L), lambda i: (0, 0)),
            pl.BlockSpec((1, _NE), lambda i: (0, 0)),
        ],
        out_specs=[
            pl.BlockSpec((T, _NE), lambda i: (0, 0)),
            pl.BlockSpec((T, 1), lambda i: (0, 0)),
            pl.BlockSpec((T, 1), lambda i: (0, 0)),
            pl.BlockSpec((T, 1), lambda i: (0, 0)),
            pl.BlockSpec((T, 1), lambda i: (0, 0)),
        ],
        out_shape=[
            jax.ShapeDtypeStruct((T, _NE), f32),
            jax.ShapeDtypeStruct((T, 1), jnp.int32),
            jax.ShapeDtypeStruct((T, 1), jnp.int32),
            jax.ShapeDtypeStruct((T, 1), f32),
            jax.ShapeDtypeStruct((T, 1), f32),
        ],
    )(x2d, ce, d1, d2, d3, d4, router_w, bias)


def _gmm_body(bexp_ref, bact_ref, xs_ref, gs_ref, w1_ref, b1_ref, w2_ref,
              b2_ref, ys_ref):
    b = pl.program_id(0)

    @pl.when(bact_ref[b] != 0)
    def _():
        xb = xs_ref[...]
        h = jax.lax.dot_general(xb, w1_ref[0], (((1,), (1,)), ((), ())),
                                preferred_element_type=jnp.float32)
        h = _gelu(h + b1_ref[0])
        o = jax.lax.dot_general(h, w2_ref[0], (((1,), (1,)), ((), ())),
                                preferred_element_type=jnp.float32)
        o = o + b2_ref[0]
        ys_ref[...] = o * gs_ref[:, 0:1]


def _run_gmm(xs, gs2d, bexp, bact, c_fc_w, c_fc_b, c_proj_w, c_proj_b):
    grid_spec = pltpu.PrefetchScalarGridSpec(
        num_scalar_prefetch=2,
        grid=(_NB,),
        in_specs=[
            pl.BlockSpec((_BT, _C), lambda b, be, ba: (b, 0)),
            pl.BlockSpec((_BT, 128), lambda b, be, ba: (b, 0)),
            pl.BlockSpec((1, 4 * _C, _C), lambda b, be, ba: (be[b], 0, 0)),
            pl.BlockSpec((1, 1, 4 * _C), lambda b, be, ba: (be[b], 0, 0)),
            pl.BlockSpec((1, _C, 4 * _C), lambda b, be, ba: (be[b], 0, 0)),
            pl.BlockSpec((1, 1, _C), lambda b, be, ba: (be[b], 0, 0)),
        ],
        out_specs=pl.BlockSpec((_BT, _C), lambda b, be, ba: (b, 0)),
    )
    return pl.pallas_call(
        _gmm_body,
        grid_spec=grid_spec,
        out_shape=jax.ShapeDtypeStruct((_NPAD, _C), jnp.float32),
    )(bexp, bact, xs, gs2d, c_fc_w, c_fc_b.reshape(_NE, 1, 4 * _C),
      c_proj_w, c_proj_b.reshape(_NE, 1, _C))


_NW = 32
_PPW = 4096 // _NW


def _sc_dispatch_body(x_hbm, slot_hbm, ge_hbm, xs_hbm, gs_hbm,
                      tv0, tv1, sv0, sv1, gb, ba, bb, si, sg, ss):
    wid = lax.axis_index("s") * 2 + lax.axis_index("c")
    base = wid * _PPW
    tbase = jnp.where(wid < 16, base, base - 2048)
    ci = pltpu.async_copy(slot_hbm.at[pl.ds(base, 64)], sv0, si)
    ci2 = pltpu.async_copy(slot_hbm.at[pl.ds(base + 64, 64)], sv1, si)
    cg = pltpu.async_copy(ge_hbm.at[pl.ds(base, _PPW)], gb, si)
    iota = lax.iota(jnp.int32, 16)
    for j in range(4):
        tv0[pl.ds(j * 16, 16)] = iota + (tbase + j * 16)
        tv1[pl.ds(j * 16, 16)] = iota + (tbase + 64 + j * 16)
    c0 = pltpu.async_copy(x_hbm.at[tv0], ba, sg)
    c1 = pltpu.async_copy(x_hbm.at[tv1], bb, sg)
    ci.wait()
    ci2.wait()
    cg.wait()
    c0.wait()
    w0 = pltpu.async_copy(ba, xs_hbm.at[sv0], ss)
    c1.wait()
    w1 = pltpu.async_copy(bb, xs_hbm.at[sv1], ss)
    w2 = pltpu.async_copy(gb.at[pl.ds(0, 64)], gs_hbm.at[sv0], ss)
    w3 = pltpu.async_copy(gb.at[pl.ds(64, 64)], gs_hbm.at[sv1], ss)
    w0.wait()
    w1.wait()
    w2.wait()
    w3.wait()


def _sc_dispatch(x2d, slot, ge):
    f32 = jnp.float32
    k = pl.kernel(
        _sc_dispatch_body,
        out_type=[
            jax.ShapeDtypeStruct((_NPAD, _C), f32),
            jax.ShapeDtypeStruct((_NPAD, 128), f32),
        ],
        mesh=plsc.VectorSubcoreMesh(core_axis_name="c", subcore_axis_name="s"),
        scratch_types=[
            pltpu.VMEM((64,), jnp.int32),
            pltpu.VMEM((64,), jnp.int32),
            pltpu.VMEM((64,), jnp.int32),
            pltpu.VMEM((64,), jnp.int32),
            pltpu.VMEM((_PPW, 128), f32),
            pltpu.VMEM((64, _C), f32),
            pltpu.VMEM((64, _C), f32),
            pltpu.SemaphoreType.DMA,
            pltpu.SemaphoreType.DMA,
            pltpu.SemaphoreType.DMA,
        ],
    )
    return k(x2d, slot, ge)


_TPW = 2048 // _NW


def _sc_combine_body(ys_hbm, p1_hbm, p2_hbm, out_hbm,
                     ia0, ia1, ib0, ib1, ba, bb, sa, sb):
    wid = lax.axis_index("s") * 2 + lax.axis_index("c")
    base = wid * _TPW
    pltpu.sync_copy(p1_hbm.at[pl.ds(base, 32)], ia0)
    pltpu.sync_copy(p1_hbm.at[pl.ds(base + 32, 32)], ia1)
    pltpu.sync_copy(p2_hbm.at[pl.ds(base, 32)], ib0)
    pltpu.sync_copy(p2_hbm.at[pl.ds(base + 32, 32)], ib1)
    for c, (iA, iB) in enumerate(((ia0, ib0), (ia1, ib1))):
        ca = pltpu.async_copy(ys_hbm.at[iA], ba, sa)
        cb = pltpu.async_copy(ys_hbm.at[iB], bb, sb)
        ca.wait()
        cb.wait()

        def _add(r, _):
            for u in range(48):
                ba[r, pl.ds(u * 16, 16)] = (ba[r, pl.ds(u * 16, 16)]
                                            + bb[r, pl.ds(u * 16, 16)])
            return 0

        lax.fori_loop(0, 32, _add, 0)
        pltpu.sync_copy(ba, out_hbm.at[pl.ds(base + c * 32, 32)])


def _sc_combine(ys, pos1, pos2):
    f32 = jnp.float32
    k = pl.kernel(
        _sc_combine_body,
        out_type=jax.ShapeDtypeStruct((2048, _C), f32),
        mesh=plsc.VectorSubcoreMesh(core_axis_name="c", subcore_axis_name="s"),
        scratch_types=[
            pltpu.VMEM((32,), jnp.int32),
            pltpu.VMEM((32,), jnp.int32),
            pltpu.VMEM((32,), jnp.int32),
            pltpu.VMEM((32,), jnp.int32),
            pltpu.VMEM((32, _C), f32),
            pltpu.VMEM((32, _C), f32),
            pltpu.SemaphoreType.DMA,
            pltpu.SemaphoreType.DMA,
        ],
    )
    return k(ys, pos1, pos2)


def kernel(x, delta_t_info, delta_dis_info, delta_rg_info, delta_entropy_info,
           city_embeddings, router_w, router_b, c_fc_w, c_fc_b, c_proj_w,
           c_proj_b, city):
    B, T, C = x.shape
    x2d = x.reshape(T, C)
    ce = city_embeddings[city].reshape(1, 32)
    bias = router_b.reshape(1, _NE)

    gate1, i1, i2, g1, g2 = _run_router(
        x2d, ce, delta_t_info.reshape(T, -1), delta_dis_info.reshape(T, -1),
        delta_rg_info.reshape(T, -1), delta_entropy_info.reshape(T, -1),
        router_w, bias)

    e_all = jnp.concatenate([i1[:, 0], i2[:, 0]])
    g_all = jnp.concatenate([g1[:, 0], g2[:, 0]])
    oh = (e_all[:, None] == jnp.arange(_NE)[None, :]).astype(jnp.int32)
    csum = jnp.cumsum(oh, axis=0)
    rank = jnp.take_along_axis(csum, e_all[:, None], axis=1)[:, 0] - 1
    counts = csum[-1]
    nblk = (counts + _BT - 1) // _BT
    pstart_blk = jnp.concatenate(
        [jnp.zeros(1, nblk.dtype), jnp.cumsum(nblk)])
    slot = pstart_blk[e_all] * _BT + rank
    ge = jnp.broadcast_to(g_all[:, None], (2 * T, 128))
    total_blk = pstart_blk[-1]
    bidx = jnp.arange(_NB)
    bexp = jnp.searchsorted(pstart_blk[1:], bidx, side='right').astype(jnp.int32)
    park = jnp.searchsorted(pstart_blk[1:], total_blk - 1,
                            side='right').astype(jnp.int32)
    bact = (bidx < total_blk).astype(jnp.int32)
    bexp = jnp.where(bact == 1, bexp, park)

    xs, gsl16 = _sc_dispatch(x2d, slot, ge)

    ys = _run_gmm(xs, gsl16, bexp, bact,
                  c_fc_w, c_fc_b, c_proj_w, c_proj_b)

    out2d = _sc_combine(ys, slot[:T], slot[T:])

    return out2d.reshape(B, T, C), gate1.reshape(B, T, _NE)


# speedup vs baseline: 1.3440x; 1.3440x over previous
import functools

import jax
import jax.numpy as jnp
from jax import lax
from jax.experimental import pallas as pl
from jax.experimental.pallas import tpu as pltpu
from jax.experimental.pallas import tpu_sc as plsc

_C = 768
_NE = 8
_BT = 256
_NB = 23
_NPAD = _NB * _BT
_GC = 0.7978845608028654


def _gelu(x):
    return 0.5 * x * (1.0 + jnp.tanh(_GC * (x + 0.044715 * x * x * x)))


def _router_body(x_ref, ce_ref, d1_ref, d2_ref, d3_ref, d4_ref, w_ref, b_ref,
                 gate1_ref, i1_ref, i2_ref, g1_ref, g2_ref):
    T = x_ref.shape[0]
    ce = jnp.broadcast_to(ce_ref[...], (T, 32))
    h = jnp.concatenate([x_ref[...], ce, d1_ref[...], d2_ref[...],
                         d3_ref[...], d4_ref[...]], axis=1)
    l = jax.lax.dot_general(h, w_ref[...], (((1,), (1,)), ((), ())),
                            preferred_element_type=jnp.float32)
    l = l + b_ref[...]

    m = jnp.max(l, axis=1, keepdims=True)
    ex = jnp.exp(l - m)
    gate1_ref[...] = ex / jnp.sum(ex, axis=1, keepdims=True)

    T = l.shape[0]
    iota = jax.lax.broadcasted_iota(jnp.int32, (T, _NE), 1)
    m1 = jnp.max(l, axis=1, keepdims=True)
    i1 = jnp.min(jnp.where(l == m1, iota, _NE), axis=1, keepdims=True)
    l2 = jnp.where(iota == i1, -jnp.inf, l)
    m2 = jnp.max(l2, axis=1, keepdims=True)
    i2 = jnp.min(jnp.where(l2 == m2, iota, _NE), axis=1, keepdims=True)
    e21 = jnp.exp(m2 - m1)
    den = 1.0 + e21
    i1_ref[...] = i1
    i2_ref[...] = i2
    g1_ref[...] = 1.0 / den
    g2_ref[...] = e21 / den


def _run_router(x2d, ce, d1, d2, d3, d4, router_w, bias):
    T = x2d.shape[0]
    L = router_w.shape[1]
    f32 = jnp.float32
    return pl.pallas_call(
        _router_body,
        grid=(1,),
        in_specs=[
            pl.BlockSpec((T, _C), lambda i: (0, 0)),
            pl.BlockSpec((1, 32), lambda i: (0, 0)),
            pl.BlockSpec((T, 192), lambda i: (0, 0)),
            pl.BlockSpec((T, 192), lambda i: (0, 0)),
            pl.BlockSpec((T, 96), lambda i: (0, 0)),
            pl.BlockSpec((T, 96), lambda i: (0, 0)),
            pl.BlockSpec((_NE, L), lambda i: (0, 0)),
            pl.BlockSpec((1, _NE), lambda i: (0, 0)),
        ],
        out_specs=[
            pl.BlockSpec((T, _NE), lambda i: (0, 0)),
            pl.BlockSpec((T, 1), lambda i: (0, 0)),
            pl.BlockSpec((T, 1), lambda i: (0, 0)),
            pl.BlockSpec((T, 1), lambda i: (0, 0)),
            pl.BlockSpec((T, 1), lambda i: (0, 0)),
        ],
        out_shape=[
            jax.ShapeDtypeStruct((T, _NE), f32),
            jax.ShapeDtypeStruct((T, 1), jnp.int32),
            jax.ShapeDtypeStruct((T, 1), jnp.int32),
            jax.ShapeDtypeStruct((T, 1), f32),
            jax.ShapeDtypeStruct((T, 1), f32),
        ],
    )(x2d, ce, d1, d2, d3, d4, router_w, bias)


def _gmm_body(bexp_ref, bact_ref, xs_ref, gs_ref, w1_ref, b1_ref, w2_ref,
              b2_ref, ys_ref):
    b = pl.program_id(0)

    @pl.when(bact_ref[b] != 0)
    def _():
        xb = xs_ref[...]
        h = jax.lax.dot_general(xb, w1_ref[0], (((1,), (1,)), ((), ())),
                                preferred_element_type=jnp.float32)
        h = _gelu(h + b1_ref[0])
        o = jax.lax.dot_general(h, w2_ref[0], (((1,), (1,)), ((), ())),
                                preferred_element_type=jnp.float32)
        o = o + b2_ref[0]
        ys_ref[...] = o * gs_ref[:, 0:1]


def _run_gmm(xs, gs2d, bexp, bact, c_fc_w, c_fc_b, c_proj_w, c_proj_b):
    grid_spec = pltpu.PrefetchScalarGridSpec(
        num_scalar_prefetch=2,
        grid=(_NB,),
        in_specs=[
            pl.BlockSpec((_BT, _C), lambda b, be, ba: (b, 0)),
            pl.BlockSpec((_BT, 128), lambda b, be, ba: (b, 0)),
            pl.BlockSpec((1, 4 * _C, _C), lambda b, be, ba: (be[b], 0, 0)),
            pl.BlockSpec((1, 1, 4 * _C), lambda b, be, ba: (be[b], 0, 0)),
            pl.BlockSpec((1, _C, 4 * _C), lambda b, be, ba: (be[b], 0, 0)),
            pl.BlockSpec((1, 1, _C), lambda b, be, ba: (be[b], 0, 0)),
        ],
        out_specs=pl.BlockSpec((_BT, _C), lambda b, be, ba: (b, 0)),
    )
    return pl.pallas_call(
        _gmm_body,
        grid_spec=grid_spec,
        out_shape=jax.ShapeDtypeStruct((_NPAD, _C), jnp.float32),
    )(bexp, bact, xs, gs2d, c_fc_w, c_fc_b.reshape(_NE, 1, 4 * _C),
      c_proj_w, c_proj_b.reshape(_NE, 1, _C))


_NW = 32
_PPW = 4096 // _NW


def _sc_dispatch_body(x_hbm, slot_hbm, ge_hbm, xs_hbm, gs_hbm,
                      tv0, tv1, sv0, sv1, gb, ba, bb, si, sg, ss):
    wid = lax.axis_index("s") * 2 + lax.axis_index("c")
    base = wid * _PPW
    tbase = jnp.where(wid < 16, base, base - 2048)
    ci = pltpu.async_copy(slot_hbm.at[pl.ds(base, 64)], sv0, si)
    ci2 = pltpu.async_copy(slot_hbm.at[pl.ds(base + 64, 64)], sv1, si)
    cg = pltpu.async_copy(ge_hbm.at[pl.ds(base, _PPW)], gb, si)
    iota = lax.iota(jnp.int32, 16)
    for j in range(4):
        tv0[pl.ds(j * 16, 16)] = iota + (tbase + j * 16)
        tv1[pl.ds(j * 16, 16)] = iota + (tbase + 64 + j * 16)
    c0 = pltpu.async_copy(x_hbm.at[tv0], ba, sg)
    c1 = pltpu.async_copy(x_hbm.at[tv1], bb, sg)
    ci.wait()
    ci2.wait()
    cg.wait()
    c0.wait()
    w0 = pltpu.async_copy(ba, xs_hbm.at[sv0], ss)
    c1.wait()
    w1 = pltpu.async_copy(bb, xs_hbm.at[sv1], ss)
    w2 = pltpu.async_copy(gb.at[pl.ds(0, 64)], gs_hbm.at[sv0], ss)
    w3 = pltpu.async_copy(gb.at[pl.ds(64, 64)], gs_hbm.at[sv1], ss)
    w0.wait()
    w1.wait()
    w2.wait()
    w3.wait()


def _sc_dispatch(x2d, slot, ge):
    f32 = jnp.float32
    k = pl.kernel(
        _sc_dispatch_body,
        out_type=[
            jax.ShapeDtypeStruct((_NPAD, _C), f32),
            jax.ShapeDtypeStruct((_NPAD, 128), f32),
        ],
        mesh=plsc.VectorSubcoreMesh(core_axis_name="c", subcore_axis_name="s"),
        scratch_types=[
            pltpu.VMEM((64,), jnp.int32),
            pltpu.VMEM((64,), jnp.int32),
            pltpu.VMEM((64,), jnp.int32),
            pltpu.VMEM((64,), jnp.int32),
            pltpu.VMEM((_PPW, 128), f32),
            pltpu.VMEM((64, _C), f32),
            pltpu.VMEM((64, _C), f32),
            pltpu.SemaphoreType.DMA,
            pltpu.SemaphoreType.DMA,
            pltpu.SemaphoreType.DMA,
        ],
    )
    return k(x2d, slot, ge)


_TPW = 2048 // _NW


def _sc_combine_body(ys_hbm, p1_hbm, p2_hbm, out_hbm,
                     ia0, ia1, ib0, ib1, ba, bb, sa, sb):
    wid = lax.axis_index("s") * 2 + lax.axis_index("c")
    base = wid * _TPW
    pltpu.sync_copy(p1_hbm.at[pl.ds(base, 32)], ia0)
    pltpu.sync_copy(p1_hbm.at[pl.ds(base + 32, 32)], ia1)
    pltpu.sync_copy(p2_hbm.at[pl.ds(base, 32)], ib0)
    pltpu.sync_copy(p2_hbm.at[pl.ds(base + 32, 32)], ib1)
    for c, (iA, iB) in enumerate(((ia0, ib0), (ia1, ib1))):
        ca = pltpu.async_copy(ys_hbm.at[iA], ba, sa)
        cb = pltpu.async_copy(ys_hbm.at[iB], bb, sb)
        ca.wait()
        cb.wait()

        def _add(r, _):
            for u in range(48):
                ba[r, pl.ds(u * 16, 16)] = (ba[r, pl.ds(u * 16, 16)]
                                            + bb[r, pl.ds(u * 16, 16)])
            return 0

        lax.fori_loop(0, 32, _add, 0)
        pltpu.sync_copy(ba, out_hbm.at[pl.ds(base + c * 32, 32)])


def _sc_combine(ys, pos1, pos2):
    f32 = jnp.float32
    k = pl.kernel(
        _sc_combine_body,
        out_type=jax.ShapeDtypeStruct((2048, _C), f32),
        mesh=plsc.VectorSubcoreMesh(core_axis_name="c", subcore_axis_name="s"),
        scratch_types=[
            pltpu.VMEM((32,), jnp.int32),
            pltpu.VMEM((32,), jnp.int32),
            pltpu.VMEM((32,), jnp.int32),
            pltpu.VMEM((32,), jnp.int32),
            pltpu.VMEM((32, _C), f32),
            pltpu.VMEM((32, _C), f32),
            pltpu.SemaphoreType.DMA,
            pltpu.SemaphoreType.DMA,
        ],
    )
    return k(ys, pos1, pos2)


def kernel(x, delta_t_info, delta_dis_info, delta_rg_info, delta_entropy_info,
           city_embeddings, router_w, router_b, c_fc_w, c_fc_b, c_proj_w,
           c_proj_b, city):
    B, T, C = x.shape
    x2d = x.reshape(T, C)
    ce = city_embeddings[city].reshape(1, 32)
    bias = router_b.reshape(1, _NE)

    gate1, i1, i2, g1, g2 = _run_router(
        x2d, ce, delta_t_info.reshape(T, -1), delta_dis_info.reshape(T, -1),
        delta_rg_info.reshape(T, -1), delta_entropy_info.reshape(T, -1),
        router_w, bias)

    e_all = jnp.concatenate([i1[:, 0], i2[:, 0]])
    g_all = jnp.concatenate([g1[:, 0], g2[:, 0]])
    oh = (e_all[:, None] == jnp.arange(_NE)[None, :]).astype(jnp.int32)
    csum = jnp.cumsum(oh, axis=0)
    rank = jnp.take_along_axis(csum, e_all[:, None], axis=1)[:, 0] - 1
    counts = csum[-1]
    nblk = (counts + _BT - 1) // _BT
    pstart_blk = jnp.concatenate(
        [jnp.zeros(1, nblk.dtype), jnp.cumsum(nblk)])
    slot = pstart_blk[e_all] * _BT + rank
    ge = jnp.broadcast_to(g_all[:, None], (2 * T, 128))
    total_blk = pstart_blk[-1]
    bidx = jnp.arange(_NB)
    bexp = jnp.searchsorted(pstart_blk[1:], bidx, side='right').astype(jnp.int32)
    park = jnp.searchsorted(pstart_blk[1:], total_blk - 1,
                            side='right').astype(jnp.int32)
    bact = (bidx < total_blk).astype(jnp.int32)
    bexp = jnp.where(bact == 1, bexp, park)

    xs, gsl16 = _sc_dispatch(x2d, slot, ge)

    ys = _run_gmm(xs, gsl16, bexp, bact,
                  c_fc_w, c_fc_b, c_proj_w, c_proj_b)

    out2d = _sc_combine(ys, slot[:T], slot[T:])

    return out2d.reshape(B, T, C), gate1.reshape(B, T, _NE)


# speedup vs baseline: 1.4555x; 1.0830x over previous
import functools

import jax
import jax.numpy as jnp
from jax import lax
from jax.experimental import pallas as pl
from jax.experimental.pallas import tpu as pltpu
from jax.experimental.pallas import tpu_sc as plsc

_C = 768
_NE = 8
_BT = 256
_NB = 23
_NPAD = _NB * _BT
_GC = 0.7978845608028654


def _gelu(x):
    return 0.5 * x * (1.0 + jnp.tanh(_GC * (x + 0.044715 * x * x * x)))


def _router_body(x_ref, ce_ref, d1_ref, d2_ref, d3_ref, d4_ref, w_ref, b_ref,
                 gate1_ref, slot_ref, bexp_ref, bact_ref, ge_ref, rank_ref):
    T = x_ref.shape[0]
    ce = jnp.broadcast_to(ce_ref[...], (T, 32))
    h = jnp.concatenate([x_ref[...], ce, d1_ref[...], d2_ref[...],
                         d3_ref[...], d4_ref[...]], axis=1)
    l = jax.lax.dot_general(h, w_ref[...], (((1,), (1,)), ((), ())),
                            preferred_element_type=jnp.float32)
    l = l + b_ref[...]

    m = jnp.max(l, axis=1, keepdims=True)
    ex = jnp.exp(l - m)
    gate1_ref[...] = ex / jnp.sum(ex, axis=1, keepdims=True)

    T = l.shape[0]
    iota = jax.lax.broadcasted_iota(jnp.int32, (T, _NE), 1)
    m1 = jnp.max(l, axis=1, keepdims=True)
    i1 = jnp.min(jnp.where(l == m1, iota, _NE), axis=1, keepdims=True)
    l2 = jnp.where(iota == i1, -jnp.inf, l)
    m2 = jnp.max(l2, axis=1, keepdims=True)
    i2 = jnp.min(jnp.where(l2 == m2, iota, _NE), axis=1, keepdims=True)
    e21 = jnp.exp(m2 - m1)
    den = 1.0 + e21
    g1 = 1.0 / den
    g2 = e21 / den

    ef = jnp.concatenate([i1, i2], axis=0).astype(jnp.float32)
    gall = jnp.concatenate([g1, g2], axis=0)
    ge_ref[...] = jnp.broadcast_to(gall, (2 * T, 128))
    eiota = jax.lax.broadcasted_iota(jnp.int32, (2 * T, _NE), 1).astype(jnp.float32)
    oh = jnp.where(ef == eiota, 1.0, 0.0)
    ck = 256
    nchunk = (2 * T) // ck
    ri = jax.lax.broadcasted_iota(jnp.int32, (ck, ck), 0)
    cj = jax.lax.broadcasted_iota(jnp.int32, (ck, ck), 1)
    tril = jnp.where(ri >= cj, 1.0, 0.0)
    off = jnp.zeros((1, _NE), jnp.float32)
    for c in range(nchunk):
        ohc = oh[c * ck:(c + 1) * ck, :]
        csumc = jax.lax.dot_general(tril, ohc, (((1,), (0,)), ((), ())),
                                    preferred_element_type=jnp.float32)
        rank_c = (jnp.sum((csumc + off) * ohc, axis=1, keepdims=True) - 1.0)
        rank_ref[c * ck:(c + 1) * ck, :] = rank_c
        off = off + csumc[ck - 1:ck, :]
    counts = off
    nblk = jnp.floor((counts + (_BT - 1)) / _BT)
    uiota_r = jax.lax.broadcasted_iota(jnp.int32, (_NE, _NE), 0)
    uiota_c = jax.lax.broadcasted_iota(jnp.int32, (_NE, _NE), 1)
    ustrict = jnp.where(uiota_r < uiota_c, 1.0, 0.0)
    pstartb = jax.lax.dot_general(nblk, ustrict, (((1,), (0,)), ((), ())),
                                  preferred_element_type=jnp.float32)
    ends = pstartb + nblk
    total_blk = jnp.sum(nblk, axis=1, keepdims=True)
    for c in range(nchunk):
        ohc = oh[c * ck:(c + 1) * ck, :]
        ps = jnp.sum(ohc * pstartb, axis=1, keepdims=True) * _BT
        slot_ref[c * ck:(c + 1) * ck, :] = (
            ps + rank_ref[c * ck:(c + 1) * ck, :]).astype(jnp.int32)
    biota = jax.lax.broadcasted_iota(jnp.int32, (_NB, 1), 0).astype(jnp.float32)
    bexp = jnp.sum(jnp.where(ends <= biota, 1.0, 0.0), axis=1, keepdims=True)
    park = jnp.sum(jnp.where(ends <= total_blk - 1.0, 1.0, 0.0),
                   axis=1, keepdims=True)
    bact = biota < total_blk
    bexp_ref[...] = jnp.where(bact, bexp, park).astype(jnp.int32)
    bact_ref[...] = bact.astype(jnp.int32)


def _run_router(x2d, ce, d1, d2, d3, d4, router_w, bias):
    T = x2d.shape[0]
    L = router_w.shape[1]
    f32 = jnp.float32
    return pl.pallas_call(
        _router_body,
        grid=(1,),
        in_specs=[
            pl.BlockSpec((T, _C), lambda i: (0, 0)),
            pl.BlockSpec((1, 32), lambda i: (0, 0)),
            pl.BlockSpec((T, 192), lambda i: (0, 0)),
            pl.BlockSpec((T, 192), lambda i: (0, 0)),
            pl.BlockSpec((T, 96), lambda i: (0, 0)),
            pl.BlockSpec((T, 96), lambda i: (0, 0)),
            pl.BlockSpec((_NE, L), lambda i: (0, 0)),
            pl.BlockSpec((1, _NE), lambda i: (0, 0)),
        ],
        out_specs=[
            pl.BlockSpec((T, _NE), lambda i: (0, 0)),
            pl.BlockSpec((2 * T, 1), lambda i: (0, 0)),
            pl.BlockSpec((_NB, 1), lambda i: (0, 0)),
            pl.BlockSpec((_NB, 1), lambda i: (0, 0)),
            pl.BlockSpec((2 * T, 128), lambda i: (0, 0)),
        ],
        out_shape=[
            jax.ShapeDtypeStruct((T, _NE), f32),
            jax.ShapeDtypeStruct((2 * T, 1), jnp.int32),
            jax.ShapeDtypeStruct((_NB, 1), jnp.int32),
            jax.ShapeDtypeStruct((_NB, 1), jnp.int32),
            jax.ShapeDtypeStruct((2 * T, 128), f32),
        ],
        scratch_shapes=[pltpu.VMEM((2 * T, 1), f32)],
    )(x2d, ce, d1, d2, d3, d4, router_w, bias)


def _gmm_body(bexp_ref, bact_ref, xs_ref, gs_ref, w1_ref, b1_ref, w2_ref,
              b2_ref, ys_ref):
    b = pl.program_id(0)

    @pl.when(bact_ref[b] != 0)
    def _():
        xb = xs_ref[...]
        h = jax.lax.dot_general(xb, w1_ref[0], (((1,), (1,)), ((), ())),
                                preferred_element_type=jnp.float32)
        h = _gelu(h + b1_ref[0])
        o = jax.lax.dot_general(h, w2_ref[0], (((1,), (1,)), ((), ())),
                                preferred_element_type=jnp.float32)
        o = o + b2_ref[0]
        ys_ref[...] = o * gs_ref[:, 0:1]


def _run_gmm(xs, gs2d, bexp, bact, c_fc_w, c_fc_b, c_proj_w, c_proj_b):
    grid_spec = pltpu.PrefetchScalarGridSpec(
        num_scalar_prefetch=2,
        grid=(_NB,),
        in_specs=[
            pl.BlockSpec((_BT, _C), lambda b, be, ba: (b, 0)),
            pl.BlockSpec((_BT, 128), lambda b, be, ba: (b, 0)),
            pl.BlockSpec((1, 4 * _C, _C), lambda b, be, ba: (be[b], 0, 0)),
            pl.BlockSpec((1, 1, 4 * _C), lambda b, be, ba: (be[b], 0, 0)),
            pl.BlockSpec((1, _C, 4 * _C), lambda b, be, ba: (be[b], 0, 0)),
            pl.BlockSpec((1, 1, _C), lambda b, be, ba: (be[b], 0, 0)),
        ],
        out_specs=pl.BlockSpec((_BT, _C), lambda b, be, ba: (b, 0)),
    )
    return pl.pallas_call(
        _gmm_body,
        grid_spec=grid_spec,
        out_shape=jax.ShapeDtypeStruct((_NPAD, _C), jnp.float32),
    )(bexp, bact, xs, gs2d, c_fc_w, c_fc_b.reshape(_NE, 1, 4 * _C),
      c_proj_w, c_proj_b.reshape(_NE, 1, _C))


_NW = 32
_PPW = 4096 // _NW


def _sc_dispatch_body(x_hbm, slot_hbm, ge_hbm, xs_hbm, gs_hbm,
                      tv0, tv1, sv0, sv1, gb, ba, bb, si, sg, ss):
    wid = lax.axis_index("s") * 2 + lax.axis_index("c")
    base = wid * _PPW
    tbase = jnp.where(wid < 16, base, base - 2048)
    ci = pltpu.async_copy(slot_hbm.at[pl.ds(base, 64)], sv0, si)
    ci2 = pltpu.async_copy(slot_hbm.at[pl.ds(base + 64, 64)], sv1, si)
    cg = pltpu.async_copy(ge_hbm.at[pl.ds(base, _PPW)], gb, si)
    iota = lax.iota(jnp.int32, 16)
    for j in range(4):
        tv0[pl.ds(j * 16, 16)] = iota + (tbase + j * 16)
        tv1[pl.ds(j * 16, 16)] = iota + (tbase + 64 + j * 16)
    c0 = pltpu.async_copy(x_hbm.at[tv0], ba, sg)
    c1 = pltpu.async_copy(x_hbm.at[tv1], bb, sg)
    ci.wait()
    ci2.wait()
    cg.wait()
    c0.wait()
    w0 = pltpu.async_copy(ba, xs_hbm.at[sv0], ss)
    c1.wait()
    w1 = pltpu.async_copy(bb, xs_hbm.at[sv1], ss)
    w2 = pltpu.async_copy(gb.at[pl.ds(0, 64)], gs_hbm.at[sv0], ss)
    w3 = pltpu.async_copy(gb.at[pl.ds(64, 64)], gs_hbm.at[sv1], ss)
    w0.wait()
    w1.wait()
    w2.wait()
    w3.wait()


def _sc_dispatch(x2d, slot, ge):
    f32 = jnp.float32
    k = pl.kernel(
        _sc_dispatch_body,
        out_type=[
            jax.ShapeDtypeStruct((_NPAD, _C), f32),
            jax.ShapeDtypeStruct((_NPAD, 128), f32),
        ],
        mesh=plsc.VectorSubcoreMesh(core_axis_name="c", subcore_axis_name="s"),
        scratch_types=[
            pltpu.VMEM((64,), jnp.int32),
            pltpu.VMEM((64,), jnp.int32),
            pltpu.VMEM((64,), jnp.int32),
            pltpu.VMEM((64,), jnp.int32),
            pltpu.VMEM((_PPW, 128), f32),
            pltpu.VMEM((64, _C), f32),
            pltpu.VMEM((64, _C), f32),
            pltpu.SemaphoreType.DMA,
            pltpu.SemaphoreType.DMA,
            pltpu.SemaphoreType.DMA,
        ],
    )
    return k(x2d, slot, ge)


_TPW = 2048 // _NW


def _sc_combine_body(ys_hbm, p1_hbm, p2_hbm, out_hbm,
                     ia0, ia1, ib0, ib1, ba, bb, sa, sb):
    wid = lax.axis_index("s") * 2 + lax.axis_index("c")
    base = wid * _TPW
    pltpu.sync_copy(p1_hbm.at[pl.ds(base, 32)], ia0)
    pltpu.sync_copy(p1_hbm.at[pl.ds(base + 32, 32)], ia1)
    pltpu.sync_copy(p2_hbm.at[pl.ds(base, 32)], ib0)
    pltpu.sync_copy(p2_hbm.at[pl.ds(base + 32, 32)], ib1)
    for c, (iA, iB) in enumerate(((ia0, ib0), (ia1, ib1))):
        ca = pltpu.async_copy(ys_hbm.at[iA], ba, sa)
        cb = pltpu.async_copy(ys_hbm.at[iB], bb, sb)
        ca.wait()
        cb.wait()

        def _add(r, _):
            for u in range(48):
                ba[r, pl.ds(u * 16, 16)] = (ba[r, pl.ds(u * 16, 16)]
                                            + bb[r, pl.ds(u * 16, 16)])
            return 0

        lax.fori_loop(0, 32, _add, 0)
        pltpu.sync_copy(ba, out_hbm.at[pl.ds(base + c * 32, 32)])


def _sc_combine(ys, pos1, pos2):
    f32 = jnp.float32
    k = pl.kernel(
        _sc_combine_body,
        out_type=jax.ShapeDtypeStruct((2048, _C), f32),
        mesh=plsc.VectorSubcoreMesh(core_axis_name="c", subcore_axis_name="s"),
        scratch_types=[
            pltpu.VMEM((32,), jnp.int32),
            pltpu.VMEM((32,), jnp.int32),
            pltpu.VMEM((32,), jnp.int32),
            pltpu.VMEM((32,), jnp.int32),
            pltpu.VMEM((32, _C), f32),
            pltpu.VMEM((32, _C), f32),
            pltpu.SemaphoreType.DMA,
            pltpu.SemaphoreType.DMA,
        ],
    )
    return k(ys, pos1, pos2)


def kernel(x, delta_t_info, delta_dis_info, delta_rg_info, delta_entropy_info,
           city_embeddings, router_w, router_b, c_fc_w, c_fc_b, c_proj_w,
           c_proj_b, city):
    B, T, C = x.shape
    x2d = x.reshape(T, C)
    ce = city_embeddings[city].reshape(1, 32)
    bias = router_b.reshape(1, _NE)

    gate1, slot2d, bexp2d, bact2d, ge = _run_router(
        x2d, ce, delta_t_info.reshape(T, -1), delta_dis_info.reshape(T, -1),
        delta_rg_info.reshape(T, -1), delta_entropy_info.reshape(T, -1),
        router_w, bias)
    slot = slot2d[:, 0]

    xs, gsl16 = _sc_dispatch(x2d, slot, ge)

    ys = _run_gmm(xs, gsl16, bexp2d[:, 0], bact2d[:, 0],
                  c_fc_w, c_fc_b, c_proj_w, c_proj_b)

    out2d = _sc_combine(ys, slot[:T], slot[T:])

    return out2d.reshape(B, T, C), gate1.reshape(B, T, _NE)


# speedup vs baseline: 1.4780x; 1.0154x over previous
import functools

import jax
import jax.numpy as jnp
from jax import lax
from jax.experimental import pallas as pl
from jax.experimental.pallas import tpu as pltpu
from jax.experimental.pallas import tpu_sc as plsc

_C = 768
_NE = 8
_BT = 256
_NB = 23
_NPAD = _NB * _BT
_GC = 0.7978845608028654


def _gelu(x):
    return 0.5 * x * (1.0 + jnp.tanh(_GC * (x + 0.044715 * x * x * x)))


def _router_body(x_ref, ce_ref, d1_ref, d2_ref, d3_ref, d4_ref, w_ref, b_ref,
                 gate1_ref, slot_ref, bexp_ref, bact_ref, ge_ref, rank_ref):
    T = x_ref.shape[0]
    ce = jnp.broadcast_to(ce_ref[...], (T, 32))
    h = jnp.concatenate([x_ref[...], ce, d1_ref[...], d2_ref[...],
                         d3_ref[...], d4_ref[...]], axis=1)
    l = jax.lax.dot_general(h, w_ref[...], (((1,), (1,)), ((), ())),
                            preferred_element_type=jnp.float32)
    l = l + b_ref[...]

    m = jnp.max(l, axis=1, keepdims=True)
    ex = jnp.exp(l - m)
    gate1_ref[...] = ex / jnp.sum(ex, axis=1, keepdims=True)

    T = l.shape[0]
    iota = jax.lax.broadcasted_iota(jnp.int32, (T, _NE), 1)
    m1 = jnp.max(l, axis=1, keepdims=True)
    i1 = jnp.min(jnp.where(l == m1, iota, _NE), axis=1, keepdims=True)
    l2 = jnp.where(iota == i1, -jnp.inf, l)
    m2 = jnp.max(l2, axis=1, keepdims=True)
    i2 = jnp.min(jnp.where(l2 == m2, iota, _NE), axis=1, keepdims=True)
    e21 = jnp.exp(m2 - m1)
    den = 1.0 + e21
    g1 = 1.0 / den
    g2 = e21 / den

    ef = jnp.concatenate([i1, i2], axis=0).astype(jnp.float32)
    gall = jnp.concatenate([g1, g2], axis=0)
    ge_ref[...] = jnp.broadcast_to(gall, (2 * T, 128))
    eiota = jax.lax.broadcasted_iota(jnp.int32, (2 * T, _NE), 1).astype(jnp.float32)
    oh = jnp.where(ef == eiota, 1.0, 0.0)
    ck = 256
    nchunk = (2 * T) // ck
    ri = jax.lax.broadcasted_iota(jnp.int32, (ck, ck), 0)
    cj = jax.lax.broadcasted_iota(jnp.int32, (ck, ck), 1)
    tril = jnp.where(ri >= cj, 1.0, 0.0)
    off = jnp.zeros((1, _NE), jnp.float32)
    for c in range(nchunk):
        ohc = oh[c * ck:(c + 1) * ck, :]
        csumc = jax.lax.dot_general(tril, ohc, (((1,), (0,)), ((), ())),
                                    preferred_element_type=jnp.float32)
        rank_c = (jnp.sum((csumc + off) * ohc, axis=1, keepdims=True) - 1.0)
        rank_ref[c * ck:(c + 1) * ck, :] = rank_c
        off = off + csumc[ck - 1:ck, :]
    counts = off
    nblk = jnp.floor((counts + (_BT - 1)) / _BT)
    uiota_r = jax.lax.broadcasted_iota(jnp.int32, (_NE, _NE), 0)
    uiota_c = jax.lax.broadcasted_iota(jnp.int32, (_NE, _NE), 1)
    ustrict = jnp.where(uiota_r < uiota_c, 1.0, 0.0)
    pstartb = jax.lax.dot_general(nblk, ustrict, (((1,), (0,)), ((), ())),
                                  preferred_element_type=jnp.float32)
    ends = pstartb + nblk
    total_blk = jnp.sum(nblk, axis=1, keepdims=True)
    for c in range(nchunk):
        ohc = oh[c * ck:(c + 1) * ck, :]
        ps = jnp.sum(ohc * pstartb, axis=1, keepdims=True) * _BT
        slot_ref[c * ck:(c + 1) * ck, :] = (
            ps + rank_ref[c * ck:(c + 1) * ck, :]).astype(jnp.int32)
    biota = jax.lax.broadcasted_iota(jnp.int32, (_NB, 1), 0).astype(jnp.float32)
    bexp = jnp.sum(jnp.where(ends <= biota, 1.0, 0.0), axis=1, keepdims=True)
    park = jnp.sum(jnp.where(ends <= total_blk - 1.0, 1.0, 0.0),
                   axis=1, keepdims=True)
    bact = biota < total_blk
    bexp_ref[...] = jnp.where(bact, bexp, park).astype(jnp.int32)
    bact_ref[...] = bact.astype(jnp.int32)


def _run_router(x2d, ce, d1, d2, d3, d4, router_w, bias):
    T = x2d.shape[0]
    L = router_w.shape[1]
    f32 = jnp.float32
    return pl.pallas_call(
        _router_body,
        grid=(1,),
        in_specs=[
            pl.BlockSpec((T, _C), lambda i: (0, 0)),
            pl.BlockSpec((1, 32), lambda i: (0, 0)),
            pl.BlockSpec((T, 192), lambda i: (0, 0)),
            pl.BlockSpec((T, 192), lambda i: (0, 0)),
            pl.BlockSpec((T, 96), lambda i: (0, 0)),
            pl.BlockSpec((T, 96), lambda i: (0, 0)),
            pl.BlockSpec((_NE, L), lambda i: (0, 0)),
            pl.BlockSpec((1, _NE), lambda i: (0, 0)),
        ],
        out_specs=[
            pl.BlockSpec((T, _NE), lambda i: (0, 0)),
            pl.BlockSpec((2 * T, 1), lambda i: (0, 0)),
            pl.BlockSpec((_NB, 1), lambda i: (0, 0)),
            pl.BlockSpec((_NB, 1), lambda i: (0, 0)),
            pl.BlockSpec((2 * T, 128), lambda i: (0, 0)),
        ],
        out_shape=[
            jax.ShapeDtypeStruct((T, _NE), f32),
            jax.ShapeDtypeStruct((2 * T, 1), jnp.int32),
            jax.ShapeDtypeStruct((_NB, 1), jnp.int32),
            jax.ShapeDtypeStruct((_NB, 1), jnp.int32),
            jax.ShapeDtypeStruct((2 * T, 128), f32),
        ],
        scratch_shapes=[pltpu.VMEM((2 * T, 1), f32)],
    )(x2d, ce, d1, d2, d3, d4, router_w, bias)


def _gmm_body(bexp_ref, bact_ref, xs_ref, gs_ref, w1_ref, b1_ref, w2_ref,
              b2_ref, ys_ref):
    b = pl.program_id(0)

    @pl.when(bact_ref[b] != 0)
    def _():
        xb = xs_ref[...]
        h = jax.lax.dot_general(xb, w1_ref[0], (((1,), (1,)), ((), ())),
                                preferred_element_type=jnp.float32)
        h = _gelu(h + b1_ref[0])
        o = jax.lax.dot_general(h, w2_ref[0], (((1,), (1,)), ((), ())),
                                preferred_element_type=jnp.float32)
        o = o + b2_ref[0]
        ys_ref[...] = o * gs_ref[:, 0:1]


def _run_gmm(xs, gs2d, bexp, bact, c_fc_w, c_fc_b, c_proj_w, c_proj_b):
    grid_spec = pltpu.PrefetchScalarGridSpec(
        num_scalar_prefetch=2,
        grid=(_NB,),
        in_specs=[
            pl.BlockSpec((_BT, _C), lambda b, be, ba: (b, 0)),
            pl.BlockSpec((_BT, 128), lambda b, be, ba: (b, 0)),
            pl.BlockSpec((1, 4 * _C, _C), lambda b, be, ba: (be[b], 0, 0)),
            pl.BlockSpec((1, 1, 4 * _C), lambda b, be, ba: (be[b], 0, 0)),
            pl.BlockSpec((1, _C, 4 * _C), lambda b, be, ba: (be[b], 0, 0)),
            pl.BlockSpec((1, 1, _C), lambda b, be, ba: (be[b], 0, 0)),
        ],
        out_specs=pl.BlockSpec((_BT, _C), lambda b, be, ba: (b, 0)),
    )
    return pl.pallas_call(
        _gmm_body,
        grid_spec=grid_spec,
        out_shape=jax.ShapeDtypeStruct((_NPAD, _C), jnp.float32),
    )(bexp, bact, xs, gs2d, c_fc_w, c_fc_b.reshape(_NE, 1, 4 * _C),
      c_proj_w, c_proj_b.reshape(_NE, 1, _C))


_NW = 32
_PPW = 4096 // _NW


def _sc_dispatch_body(x_hbm, slot_hbm, ge_hbm, xs_hbm, gs_hbm,
                      tv0, tv1, sv0, sv1, gb, ba, bb, si, sg, ss):
    wid = lax.axis_index("s") * 2 + lax.axis_index("c")
    base = wid * _PPW
    tbase = jnp.where(wid < 16, base, base - 2048)
    ci = pltpu.async_copy(slot_hbm.at[pl.ds(base, 64)], sv0, si)
    ci2 = pltpu.async_copy(slot_hbm.at[pl.ds(base + 64, 64)], sv1, si)
    cg = pltpu.async_copy(ge_hbm.at[pl.ds(base, _PPW)], gb, si)
    iota = lax.iota(jnp.int32, 16)
    for j in range(4):
        tv0[pl.ds(j * 16, 16)] = iota + (tbase + j * 16)
        tv1[pl.ds(j * 16, 16)] = iota + (tbase + 64 + j * 16)
    c0 = pltpu.async_copy(x_hbm.at[tv0], ba, sg)
    c1 = pltpu.async_copy(x_hbm.at[tv1], bb, sg)
    ci.wait()
    ci2.wait()
    cg.wait()
    c0.wait()
    w0 = pltpu.async_copy(ba, xs_hbm.at[sv0], ss)
    c1.wait()
    w1 = pltpu.async_copy(bb, xs_hbm.at[sv1], ss)
    w2 = pltpu.async_copy(gb.at[pl.ds(0, 64)], gs_hbm.at[sv0], ss)
    w3 = pltpu.async_copy(gb.at[pl.ds(64, 64)], gs_hbm.at[sv1], ss)
    w0.wait()
    w1.wait()
    w2.wait()
    w3.wait()


def _sc_dispatch(x2d, slot, ge):
    f32 = jnp.float32
    k = pl.kernel(
        _sc_dispatch_body,
        out_type=[
            jax.ShapeDtypeStruct((_NPAD, _C), f32),
            jax.ShapeDtypeStruct((_NPAD, 128), f32),
        ],
        mesh=plsc.VectorSubcoreMesh(core_axis_name="c", subcore_axis_name="s"),
        scratch_types=[
            pltpu.VMEM((64,), jnp.int32),
            pltpu.VMEM((64,), jnp.int32),
            pltpu.VMEM((64,), jnp.int32),
            pltpu.VMEM((64,), jnp.int32),
            pltpu.VMEM((_PPW, 128), f32),
            pltpu.VMEM((64, _C), f32),
            pltpu.VMEM((64, _C), f32),
            pltpu.SemaphoreType.DMA,
            pltpu.SemaphoreType.DMA,
            pltpu.SemaphoreType.DMA,
        ],
    )
    return k(x2d, slot, ge)


_TPW = 2048 // _NW


def _sc_combine_body(ys_hbm, slot_hbm, out_hbm,
                     ia0, ia1, ib0, ib1, ba, bb, ba2, bb2, sa, sb):
    wid = lax.axis_index("s") * 2 + lax.axis_index("c")
    base = wid * _TPW
    pltpu.sync_copy(slot_hbm.at[pl.ds(base, 32)], ia0)
    pltpu.sync_copy(slot_hbm.at[pl.ds(base + 32, 32)], ia1)
    pltpu.sync_copy(slot_hbm.at[pl.ds(2048 + base, 32)], ib0)
    pltpu.sync_copy(slot_hbm.at[pl.ds(2048 + base + 32, 32)], ib1)
    c0a = pltpu.async_copy(ys_hbm.at[ia0], ba, sa)
    c0b = pltpu.async_copy(ys_hbm.at[ib0], bb, sa)
    c1a = pltpu.async_copy(ys_hbm.at[ia1], ba2, sb)
    c1b = pltpu.async_copy(ys_hbm.at[ib1], bb2, sb)

    def _mk_add(dst, other):
        def _add(r, _):
            for u in range(48):
                dst[r, pl.ds(u * 16, 16)] = (dst[r, pl.ds(u * 16, 16)]
                                             + other[r, pl.ds(u * 16, 16)])
            return 0
        return _add

    c0a.wait()
    c0b.wait()
    lax.fori_loop(0, 32, _mk_add(ba, bb), 0)
    w0 = pltpu.async_copy(ba, out_hbm.at[pl.ds(base, 32)], sa)
    c1a.wait()
    c1b.wait()
    lax.fori_loop(0, 32, _mk_add(ba2, bb2), 0)
    w0.wait()
    pltpu.sync_copy(ba2, out_hbm.at[pl.ds(base + 32, 32)])


def _sc_combine(ys, slot):
    f32 = jnp.float32
    k = pl.kernel(
        _sc_combine_body,
        out_type=jax.ShapeDtypeStruct((2048, _C), f32),
        mesh=plsc.VectorSubcoreMesh(core_axis_name="c", subcore_axis_name="s"),
        scratch_types=[
            pltpu.VMEM((32,), jnp.int32),
            pltpu.VMEM((32,), jnp.int32),
            pltpu.VMEM((32,), jnp.int32),
            pltpu.VMEM((32,), jnp.int32),
            pltpu.VMEM((32, _C), f32),
            pltpu.VMEM((32, _C), f32),
            pltpu.VMEM((32, _C), f32),
            pltpu.VMEM((32, _C), f32),
            pltpu.SemaphoreType.DMA,
            pltpu.SemaphoreType.DMA,
        ],
    )
    return k(ys, slot)


def kernel(x, delta_t_info, delta_dis_info, delta_rg_info, delta_entropy_info,
           city_embeddings, router_w, router_b, c_fc_w, c_fc_b, c_proj_w,
           c_proj_b, city):
    B, T, C = x.shape
    x2d = x.reshape(T, C)
    ce = city_embeddings[city].reshape(1, 32)
    bias = router_b.reshape(1, _NE)

    gate1, slot2d, bexp2d, bact2d, ge = _run_router(
        x2d, ce, delta_t_info.reshape(T, -1), delta_dis_info.reshape(T, -1),
        delta_rg_info.reshape(T, -1), delta_entropy_info.reshape(T, -1),
        router_w, bias)
    slot = slot2d[:, 0]

    xs, gsl16 = _sc_dispatch(x2d, slot, ge)

    ys = _run_gmm(xs, gsl16, bexp2d[:, 0], bact2d[:, 0],
                  c_fc_w, c_fc_b, c_proj_w, c_proj_b)

    out2d = _sc_combine(ys, slot)

    return out2d.reshape(B, T, C), gate1.reshape(B, T, _NE)
